# Initial kernel scaffold; baseline (speedup 1.0000x reference)
import jax
import jax.numpy as jnp
from jax.experimental import pallas as pl


def kernel(z_src, z_dst, edge_index):
    raise NotImplementedError("write your pallas kernel here")


# speedup vs baseline: 1.0347x; 1.0347x over previous
import functools

import jax
import jax.numpy as jnp
from jax import lax
from jax.experimental import pallas as pl
from jax.experimental.pallas import tpu as pltpu
from jax.experimental.pallas import tpu_sc as plsc

_LANES = 16
_NUM_CORES = 2
_NUM_SUBCORES = 16


def _edge_scores(z_src, z_dst, src_idx, dst_idx, *, interpret=False):
    n_nodes, d = z_src.shape
    e = src_idx.shape[0]

    nw = _NUM_CORES * _NUM_SUBCORES
    assert e % nw == 0, (e, nw)
    per_w = e // nw
    chunk = 80
    assert per_w % chunk == 0, (per_w, chunk)
    n_chunks = per_w // chunk

    mesh = plsc.VectorSubcoreMesh(core_axis_name="c", subcore_axis_name="s",
                                  num_cores=_NUM_CORES,
                                  num_subcores=_NUM_SUBCORES)

    @functools.partial(
        pl.kernel,
        out_type=jax.ShapeDtypeStruct((e,), jnp.float32),
        mesh=mesh,
        scratch_types=[
            pltpu.VMEM((chunk,), jnp.int32),
            pltpu.VMEM((chunk,), jnp.int32),
            pltpu.VMEM((chunk, d), jnp.float32),
            pltpu.VMEM((chunk, d), jnp.float32),
            pltpu.VMEM((chunk,), jnp.float32),
            pltpu.SemaphoreType.DMA,
        ],
        compiler_params=pltpu.CompilerParams(needs_layout_passes=False),
        interpret=interpret,
    )
    def _k(zsrc_hbm, zdst_hbm, sidx_hbm, didx_hbm, out_hbm,
           sidx_v, didx_v, srows_v, drows_v, out_v, sem):
        wid = lax.axis_index("s") * _NUM_CORES + lax.axis_index("c")
        base = wid * per_w

        def chunk_body(g, carry):
            off = base + g * chunk
            pltpu.sync_copy(sidx_hbm.at[pl.ds(off, chunk)], sidx_v)
            pltpu.sync_copy(didx_hbm.at[pl.ds(off, chunk)], didx_v)
            pltpu.async_copy(zsrc_hbm.at[sidx_v], srows_v, sem).wait()
            pltpu.async_copy(zdst_hbm.at[didx_v], drows_v, sem).wait()

            def blk(b, bcarry):
                rows = lax.iota(jnp.int32, _LANES) + b * _LANES

                def dstep(col, acc):
                    cols = jnp.full((_LANES,), col, jnp.int32)
                    s = plsc.load_gather(srows_v, [rows, cols])
                    t = plsc.load_gather(drows_v, [rows, cols])
                    return acc + s * t

                logit = lax.fori_loop(0, d, dstep,
                                      jnp.zeros((_LANES,), jnp.float32),
                                      unroll=8)
                out_v[pl.ds(b * _LANES, _LANES)] = 1.0 / (1.0 + jnp.exp(-logit))
                return bcarry

            lax.fori_loop(0, chunk // _LANES, blk, 0)
            pltpu.sync_copy(out_v, out_hbm.at[pl.ds(off, chunk)])
            return carry

        lax.fori_loop(0, n_chunks, chunk_body, 0)

    return _k(z_src, z_dst, src_idx, dst_idx)


def kernel(z_src, z_dst, edge_index):
    ei = edge_index.astype(jnp.int32)
    return _edge_scores(z_src.astype(jnp.float32), z_dst.astype(jnp.float32),
                        ei[0], ei[1])


# speedup vs baseline: 1.1672x; 1.1280x over previous
import functools

import jax
import jax.numpy as jnp
from jax import lax
from jax.experimental import pallas as pl
from jax.experimental.pallas import tpu as pltpu
from jax.experimental.pallas import tpu_sc as plsc

_LANES = 16
_NUM_CORES = 2
_NUM_SUBCORES = 16


def _edge_scores(z_src, z_dst, src_idx, dst_idx, *, interpret=False):
    n_nodes, d = z_src.shape
    nw = _NUM_CORES * _NUM_SUBCORES
    e = src_idx.shape[0]
    assert e % nw == 0, (e, nw)
    per_w = e // nw
    chunk = 80
    assert per_w % chunk == 0, (per_w, chunk)
    n_chunks = per_w // chunk
    n_blocks = chunk // _LANES

    src_idx = src_idx.reshape(nw, n_chunks, chunk)
    dst_idx = dst_idx.reshape(nw, n_chunks, chunk)

    mesh = plsc.VectorSubcoreMesh(core_axis_name="c", subcore_axis_name="s",
                                  num_cores=_NUM_CORES,
                                  num_subcores=_NUM_SUBCORES)

    @functools.partial(
        pl.kernel,
        out_type=jax.ShapeDtypeStruct((nw, per_w), jnp.float32),
        mesh=mesh,
        scratch_types=[
            pltpu.VMEM((n_chunks, chunk), jnp.int32),
            pltpu.VMEM((n_chunks, chunk), jnp.int32),
            pltpu.VMEM((2, chunk, d), jnp.float32),
            pltpu.VMEM((2, chunk, d), jnp.float32),
            pltpu.VMEM((per_w,), jnp.float32),
            pltpu.SemaphoreType.DMA,
            pltpu.SemaphoreType.DMA,
        ],
        compiler_params=pltpu.CompilerParams(needs_layout_passes=False),
        interpret=interpret,
    )
    def _k(zsrc_hbm, zdst_hbm, sidx_hbm, didx_hbm, out_hbm,
           sidx_v, didx_v, srows_v, drows_v, out_v, sem_s, sem_d):
        wid = lax.axis_index("s") * _NUM_CORES + lax.axis_index("c")
        pltpu.sync_copy(sidx_hbm.at[wid], sidx_v)
        pltpu.sync_copy(didx_hbm.at[wid], didx_v)

        def issue(g, b):
            pltpu.async_copy(zsrc_hbm.at[sidx_v.at[g]], srows_v.at[b], sem_s)
            pltpu.async_copy(zdst_hbm.at[didx_v.at[g]], drows_v.at[b], sem_d)

        def drain(g, b):
            pltpu.make_async_copy(zsrc_hbm.at[sidx_v.at[g]], srows_v.at[b],
                                  sem_s).wait()
            pltpu.make_async_copy(zdst_hbm.at[didx_v.at[g]], drows_v.at[b],
                                  sem_d).wait()

        def compute(g, b):
            sref, dref = srows_v.at[b], drows_v.at[b]

            def blk(blk_i, carry):
                rows = lax.iota(jnp.int32, _LANES) + blk_i * _LANES
                accs = [jnp.zeros((_LANES,), jnp.float32) for _ in range(4)]
                for col in range(d):
                    cols = jnp.full((_LANES,), col, jnp.int32)
                    s = plsc.load_gather(sref, [rows, cols])
                    t = plsc.load_gather(dref, [rows, cols])
                    accs[col % 4] = accs[col % 4] + s * t
                logit = (accs[0] + accs[1]) + (accs[2] + accs[3])
                out_v[pl.ds(g * chunk + blk_i * _LANES, _LANES)] = (
                    1.0 / (1.0 + jnp.exp(-logit)))
                return carry

            lax.fori_loop(0, n_blocks, blk, 0)

        issue(0, 0)

        def pair(k, carry):
            for b in (0, 1):
                g = 2 * k + b

                @pl.when(g < n_chunks)
                def _body():
                    @pl.when(g + 1 < n_chunks)
                    def _prefetch():
                        issue(g + 1, 1 - b)

                    drain(g, b)
                    compute(g, b)

            return carry

        lax.fori_loop(0, (n_chunks + 2) // 2, pair, 0)
        pltpu.sync_copy(out_v, out_hbm.at[wid])

    out = _k(z_src, z_dst, src_idx, dst_idx)
    return out.reshape(e)


def kernel(z_src, z_dst, edge_index):
    ei = edge_index.astype(jnp.int32)
    return _edge_scores(z_src.astype(jnp.float32), z_dst.astype(jnp.float32),
                        ei[0], ei[1])


# speedup vs baseline: 5.3289x; 4.5657x over previous
import functools

import jax
import jax.numpy as jnp
from jax import lax
from jax.experimental import pallas as pl
from jax.experimental.pallas import tpu as pltpu
from jax.experimental.pallas import tpu_sc as plsc

_LANES = 16
_NUM_CORES = 2
_NUM_SUBCORES = 16


def _edge_scores(z_src_p, z_dst_p, src_idx, dst_idx, *, interpret=False):
    n_nodes, dw = z_src_p.shape
    nw = _NUM_CORES * _NUM_SUBCORES
    e = src_idx.shape[0]
    assert e % nw == 0, (e, nw)
    per_w = e // nw
    chunk = 80
    assert per_w % chunk == 0, (per_w, chunk)
    n_chunks = per_w // chunk
    n_groups = chunk // _LANES
    n_words = (dw // 2) // _LANES

    src_idx = src_idx.reshape(nw, n_chunks, chunk)
    dst_idx = dst_idx.reshape(nw, n_chunks, chunk)

    mesh = plsc.VectorSubcoreMesh(core_axis_name="c", subcore_axis_name="s",
                                  num_cores=_NUM_CORES,
                                  num_subcores=_NUM_SUBCORES)

    @functools.partial(
        pl.kernel,
        out_type=jax.ShapeDtypeStruct((nw, per_w), jnp.float32),
        mesh=mesh,
        scratch_types=[
            pltpu.VMEM((n_chunks, chunk), jnp.int32),
            pltpu.VMEM((n_chunks, chunk), jnp.int32),
            pltpu.VMEM((2, chunk, dw), jnp.int32),
            pltpu.VMEM((2, chunk, dw), jnp.int32),
            pltpu.VMEM((_LANES, _LANES), jnp.float32),
            pltpu.VMEM((per_w,), jnp.float32),
            pltpu.SemaphoreType.DMA,
            pltpu.SemaphoreType.DMA,
        ],
        compiler_params=pltpu.CompilerParams(needs_layout_passes=False),
        interpret=interpret,
    )
    def _k(zsrc_hbm, zdst_hbm, sidx_hbm, didx_hbm, out_hbm,
           sidx_v, didx_v, srows_v, drows_v, tile_v, out_v, sem_s, sem_d):
        wid = lax.axis_index("s") * _NUM_CORES + lax.axis_index("c")
        pltpu.sync_copy(sidx_hbm.at[wid], sidx_v)
        pltpu.sync_copy(didx_hbm.at[wid], didx_v)

        def issue(g, b):
            pltpu.async_copy(zsrc_hbm.at[sidx_v.at[g]], srows_v.at[b], sem_s)
            pltpu.async_copy(zdst_hbm.at[didx_v.at[g]], drows_v.at[b], sem_d)

        def drain(g, b):
            pltpu.make_async_copy(zsrc_hbm.at[sidx_v.at[g]], srows_v.at[b],
                                  sem_s).wait()
            pltpu.make_async_copy(zdst_hbm.at[didx_v.at[g]], drows_v.at[b],
                                  sem_d).wait()

        def compute(g, b):
            sref, dref = srows_v.at[b], drows_v.at[b]

            def grp(grp_i, carry):
                e0 = grp_i * _LANES
                for e_loc in range(_LANES):
                    er = e0 + e_loc
                    prods = []
                    for j in range(n_words):
                        s = plsc.bitcast(sref[er, pl.ds(j * _LANES, _LANES)],
                                         jnp.bfloat16)
                        t = plsc.bitcast(dref[er, pl.ds(j * _LANES, _LANES)],
                                         jnp.bfloat16)
                        prods.append(s * t)
                    f32s = []
                    for j in range(0, n_words, 2):
                        pa, pb = plsc.unpack(prods[j] + prods[j + 1],
                                             format=plsc.PackFormat.INTERLEAVED)
                        f32s.append(pa + pb)
                    q = f32s[0]
                    for x in f32s[1:]:
                        q = q + x
                    tile_v[e_loc, :] = q

                riota = lax.iota(jnp.int32, _LANES)
                cols = [plsc.load_gather(
                            tile_v, [riota, jnp.full((_LANES,), l, jnp.int32)])
                        for l in range(_LANES)]
                while len(cols) > 1:
                    cols = [cols[i] + cols[i + 1]
                            for i in range(0, len(cols), 2)]
                r = cols[0]
                out_v[pl.ds(g * chunk + e0, _LANES)] = (
                    1.0 / (1.0 + jnp.exp(-r)))
                return carry

            lax.fori_loop(0, n_groups, grp, 0)

        issue(0, 0)

        def pair(k, carry):
            for b in (0, 1):
                g = 2 * k + b

                @pl.when(g < n_chunks)
                def _body():
                    @pl.when(g + 1 < n_chunks)
                    def _prefetch():
                        issue(g + 1, 1 - b)

                    drain(g, b)
                    compute(g, b)

            return carry

        lax.fori_loop(0, (n_chunks + 2) // 2, pair, 0)
        pltpu.sync_copy(out_v, out_hbm.at[wid])

    out = _k(z_src_p, z_dst_p, src_idx, dst_idx)
    return out.reshape(e)


def _pack_table(z):
    zb = z.astype(jnp.bfloat16)
    n, d = zb.shape
    packed = lax.bitcast_convert_type(zb.reshape(n, d // 2, 2), jnp.int32)
    return jnp.concatenate(
        [packed, jnp.zeros((n, d - d // 2), jnp.int32)], axis=1)


def kernel(z_src, z_dst, edge_index):
    ei = edge_index.astype(jnp.int32)
    return _edge_scores(_pack_table(z_src), _pack_table(z_dst), ei[0], ei[1])
